# baseline (speedup 1.0000x reference)
import functools
import math

import jax
import jax.numpy as jnp
from jax import lax
from jax.experimental import pallas as pl
from jax.experimental.pallas import tpu as pltpu
from jax.experimental.pallas import tpu_sc as plsc

_D = 256
_H = 8
_HD = 32
_N = 1024
_NE = 1024
_AR = 8
_NT = 32
_ET = 16
_NP = 1152
_NS = 16
_ES = _NE // _NS
_RS = _NP // _NS


def _gelu(x):
  return 0.5 * x * (1.0 + jax.lax.erf(x * (1.0 / math.sqrt(2.0))))


def _ln(x, g, b):
  m = jnp.mean(x, axis=-1, keepdims=True)
  v = jnp.mean((x - m) ** 2, axis=-1, keepdims=True)
  return (x - m) * jax.lax.rsqrt(v + 1e-5) * g + b


def _mm(a, b):
  return jnp.dot(a, b, preferred_element_type=jnp.float32)


def _rel(x, w):
  h = _mm(x, w['node_proj_w']) + w['node_proj_b']
  q = _mm(h, w['query_w']) + w['query_b']
  k = _mm(h, w['key_w']) + w['key_b']
  v = _mm(h, w['value_w']) + w['value_b']
  attn_out = jnp.zeros((_N, _D), jnp.float32)
  outs = []
  for hh in range(_H):
    sl = slice(hh * _HD, (hh + 1) * _HD)
    s = jax.lax.dot_general(q[:, sl], k[:, sl], (((1,), (1,)), ((), ())),
                            preferred_element_type=jnp.float32)
    s = s * (1.0 / math.sqrt(_HD))
    m = jnp.max(s, axis=-1, keepdims=True)
    e = jnp.exp(s - m)
    p = e / jnp.sum(e, axis=-1, keepdims=True)
    outs.append(_mm(p, v[:, sl]))
  attn_out = jnp.concatenate(outs, axis=-1)
  o = _mm(attn_out, w['out_proj_w']) + w['out_proj_b'] + x
  o = _ln(o, w['norm_g'], w['norm_b'])
  ff = _mm(_gelu(_mm(o, w['ff1_w']) + w['ff1_b']), w['ff2_w']) + w['ff2_b']
  return _ln(ff + o, w['ffn_g'], w['ffn_b'])


_REL_KEYS = ['node_proj_w', 'node_proj_b', 'query_w', 'query_b',
             'key_w', 'key_b', 'value_w', 'value_b',
             'out_proj_w', 'out_proj_b', 'norm_g', 'norm_b',
             'ff1_w', 'ff1_b', 'ff2_w', 'ff2_b', 'ffn_g', 'ffn_b']
_UPD_KEYS = ['upd_w1', 'upd_w2', 'upd_b', 'upd_g', 'upd_bb']


def _read_named(refs, keys):
  return {k: r[...] for k, r in zip(keys, refs)}


def _full_spec(arr):
  nd = arr.ndim
  return pl.BlockSpec(arr.shape, lambda b, _n=nd: (0,) * _n)


def _rel_weight_args(p):
  out = []
  for k in _REL_KEYS:
    a = p[k]
    out.append(a.reshape(1, -1) if a.ndim == 1 else a)
  return out


def _upd_weight_args(p):
  return [p['upd_w'][:_D], p['upd_w'][_D:], p['upd_b'].reshape(1, _D),
          p['upd_g'].reshape(1, _D), p['upd_bb'].reshape(1, _D)]


def _tc1_body(ids_ref, emb_ref, *rest):
  rel_refs, o_ref = rest[:len(_REL_KEYS)], rest[-1]
  w = _read_named(rel_refs, _REL_KEYS)
  ids = ids_ref[0]
  tid = jax.lax.broadcasted_iota(jnp.int32, (_NT, _N), 0)
  oh = jnp.where(ids == tid, 1.0, 0.0)
  x = jax.lax.dot_general(oh, emb_ref[...], (((0,), (0,)), ((), ())),
                          preferred_element_type=jnp.float32)
  o_ref[0, :_N] = _rel(x, w)
  o_ref[0, _N:] = jnp.zeros((_NP - _N, _D), jnp.float32)


def _tc1(ids, emb, p, B):
  args = [ids.reshape(B, 1, _N), emb] + _rel_weight_args(p)
  in_specs = [pl.BlockSpec((1, 1, _N), lambda b: (b, 0, 0))]
  in_specs += [_full_spec(a) for a in args[1:]]
  return pl.pallas_call(
      _tc1_body,
      grid=(B,),
      in_specs=in_specs,
      out_specs=pl.BlockSpec((1, _NP, _D), lambda b: (b, 0, 0)),
      out_shape=jax.ShapeDtypeStruct((B, _NP, _D), jnp.float32),
  )(*args)


_HYP_KEYS = ['he_etype', 'enc_w1', 'enc_w2', 'enc_b', 'enc_g', 'enc_bb',
             'upd_w1', 'upd_w2', 'upd_b', 'upd_g', 'upd_bb']


def _hyp_weight_args(p):
  return [p['he_etype'], p['enc_w'][:_D], p['enc_w'][_D:],
          p['enc_b'].reshape(1, _D), p['enc_g'].reshape(1, _D),
          p['enc_bb'].reshape(1, _D),
          p['upd_w'][:_D], p['upd_w'][_D:], p['upd_b'].reshape(1, _D),
          p['upd_g'].reshape(1, _D), p['upd_bb'].reshape(1, _D)]


def _hyp_tail(xp_ref, ps_ref, maskf_ref, mem_ref, types_ref, hyp_refs):
  hw = _read_named(hyp_refs, _HYP_KEYS)
  x = xp_ref[0, :_N]
  maskf = maskf_ref[0]
  cnt = jnp.clip(jnp.sum(maskf, axis=-1, keepdims=True), 1.0)
  pooled = ps_ref[0] / cnt
  types = types_ref[0]
  eid = jax.lax.broadcasted_iota(jnp.int32, (_ET, _NE), 0)
  eoh = jnp.where(types == eid, 1.0, 0.0)
  edge_emb = jax.lax.dot_general(eoh, hw['he_etype'],
                                 (((0,), (0,)), ((), ())),
                                 preferred_element_type=jnp.float32)
  ef = _mm(pooled, hw['enc_w1']) + _mm(edge_emb, hw['enc_w2']) + hw['enc_b']
  ef = _ln(_gelu(ef), hw['enc_g'], hw['enc_bb'])

  mem = mem_ref[0]
  nid = jax.lax.broadcasted_iota(jnp.int32, (_NE, _N), 1)
  w = jnp.zeros((_NE, _N), jnp.float32)
  for a in range(_AR):
    hit = mem[:, a:a + 1] == nid
    w = w + jnp.where(hit, maskf[:, a:a + 1], 0.0)
  counts = jax.lax.dot_general(w, jnp.ones((_NE, 1), jnp.float32),
                               (((0,), (0,)), ((), ())),
                               preferred_element_type=jnp.float32)
  counts = jnp.clip(counts, 1.0)
  nup = jax.lax.dot_general(w, ef, (((0,), (0,)), ((), ())),
                            preferred_element_type=jnp.float32) / counts
  u = _mm(x, hw['upd_w1']) + _mm(nup, hw['upd_w2']) + hw['upd_b']
  return _ln(_gelu(u), hw['upd_g'], hw['upd_bb'])


_NHYP = len(_HYP_KEYS)


def _tc2_body(xp_ref, ps_ref, maskf_ref, mem_ref, types_ref, *rest):
  hyp_refs = rest[:_NHYP]
  rel_refs = rest[_NHYP:_NHYP + len(_REL_KEYS)]
  o_ref = rest[-1]
  u = _hyp_tail(xp_ref, ps_ref, maskf_ref, mem_ref, types_ref, hyp_refs)
  w = _read_named(rel_refs, _REL_KEYS)
  o_ref[0, :_N] = _rel(u, w)
  o_ref[0, _N:] = jnp.zeros((_NP - _N, _D), jnp.float32)


def _hyp_in_specs():
  return [
      pl.BlockSpec((1, _NP, _D), lambda b: (b, 0, 0)),
      pl.BlockSpec((1, _NE, _D), lambda b: (b, 0, 0)),
      pl.BlockSpec((1, _NE, _AR), lambda b: (b, 0, 0)),
      pl.BlockSpec((1, _NE, _AR), lambda b: (b, 0, 0)),
      pl.BlockSpec((1, 1, _NE), lambda b: (b, 0, 0)),
  ]


def _tc2(xpad, ps, maskf, members, types, p_hyp, p_rel, B):
  args = ([xpad, ps, maskf, members, types.reshape(B, 1, _NE)]
          + _hyp_weight_args(p_hyp) + _rel_weight_args(p_rel))
  in_specs = _hyp_in_specs() + [_full_spec(a) for a in args[5:]]
  return pl.pallas_call(
      _tc2_body,
      grid=(B,),
      in_specs=in_specs,
      out_specs=pl.BlockSpec((1, _NP, _D), lambda b: (b, 0, 0)),
      out_shape=jax.ShapeDtypeStruct((B, _NP, _D), jnp.float32),
  )(*args)


def _tc3_body(xp_ref, ps_ref, maskf_ref, mem_ref, types_ref, *rest):
  hyp_refs = rest[:_NHYP]
  pw_ref, pb_ref, pg_ref, pbb_ref, ox_ref, og_ref = rest[_NHYP:]
  u = _hyp_tail(xp_ref, ps_ref, maskf_ref, mem_ref, types_ref, hyp_refs)
  ox_ref[0] = u
  gm = jnp.mean(u, axis=0, keepdims=True)
  gm = _mm(gm, pw_ref[...]) + pb_ref[...]
  og_ref[0] = _ln(_gelu(gm), pg_ref[...], pbb_ref[...])


def _tc3(xpad, ps, maskf, members, types, p_hyp, params, B):
  args = ([xpad, ps, maskf, members, types.reshape(B, 1, _NE)]
          + _hyp_weight_args(p_hyp)
          + [params['pool_w'], params['pool_b'].reshape(1, _D),
             params['pool_g'].reshape(1, _D), params['pool_bb'].reshape(1, _D)])
  in_specs = _hyp_in_specs() + [_full_spec(a) for a in args[5:]]
  return pl.pallas_call(
      _tc3_body,
      grid=(B,),
      in_specs=in_specs,
      out_specs=[
          pl.BlockSpec((1, _N, _D), lambda b: (b, 0, 0)),
          pl.BlockSpec((1, 1, _D), lambda b: (b, 0, 0)),
      ],
      out_shape=[
          jax.ShapeDtypeStruct((B, _N, _D), jnp.float32),
          jax.ShapeDtypeStruct((B, 1, _D), jnp.float32),
      ],
  )(*args)


_SC_MESH = plsc.VectorSubcoreMesh(core_axis_name="c", subcore_axis_name="s")
_CW = 128


def _scg_body(table_ref, gidx_ref, out_ref, idxs, rows_v, pooled_v, sem):
  c = lax.axis_index("c")
  s = lax.axis_index("s")
  base = s * _ES
  for a in range(_AR):
    off = c * (_AR * _NE) + a * _NE + base
    pltpu.sync_copy(gidx_ref.at[pl.ds(off, _ES)], idxs[a])
  pltpu.async_copy(table_ref.at[idxs[0]], pooled_v, sem).wait()
  for a in range(1, _AR):
    pltpu.async_copy(table_ref.at[idxs[a]], rows_v, sem).wait()

    def acc(r, _):
      for k2 in range(_D // 16):
        chunk = rows_v[r, pl.ds(k2 * 16, 16)]
        plsc.addupdate(pooled_v.at[r, pl.ds(k2 * 16, 16)], chunk)
      return 0

    lax.fori_loop(0, _ES, acc, 0)
  pltpu.sync_copy(pooled_v, out_ref.at[c, pl.ds(base, _ES)])


@functools.partial(
    pl.kernel,
    mesh=_SC_MESH,
    out_type=jax.ShapeDtypeStruct((2, _NE, _D), jnp.float32),
    scratch_types=[pltpu.VMEM((_ES,), jnp.int32)] * _AR + [
        pltpu.VMEM((_ES, _D), jnp.float32),
        pltpu.VMEM((_ES, _D), jnp.float32),
        pltpu.SemaphoreType.DMA,
    ],
)
def _sc_gather(table_ref, gidx_ref, out_ref, *rest):
  idxs, (rows_v, pooled_v, sem) = rest[:_AR], rest[_AR:]
  _scg_body(table_ref, gidx_ref, out_ref, idxs, rows_v, pooled_v, sem)


def kernel(node_type_ids, edge_index, edge_types, hyperedge_members,
           hyperedge_types, hyperedge_mask, params):
  del edge_index, edge_types
  B = node_type_ids.shape[0]

  ids = node_type_ids.astype(jnp.int32)
  members = hyperedge_members.astype(jnp.int32)
  types = hyperedge_types.astype(jnp.int32)
  maskf = hyperedge_mask.astype(jnp.float32)

  sidx = jnp.where(hyperedge_mask, members, _N).transpose(0, 2, 1)
  boff = (jnp.arange(B, dtype=jnp.int32) * _NP).reshape(B, 1, 1)
  gidx = (sidx + boff).reshape(-1)

  lyr = params['layers']
  x_pad = _tc1(ids, params['node_type_embed'], lyr[0], B)
  ps1 = _sc_gather(x_pad.reshape(B * _NP, _D), gidx)
  x_pad = _tc2(x_pad, ps1, maskf, members, types, lyr[0], lyr[1], B)
  ps2 = _sc_gather(x_pad.reshape(B * _NP, _D), gidx)
  x_out, graph_emb = _tc3(x_pad, ps2, maskf, members, types, lyr[1],
                          params, B)
  return x_out, graph_emb.reshape(B, _D)


# speedup vs baseline: 3.0382x; 3.0382x over previous
import math

import jax
import jax.numpy as jnp
from jax.experimental import pallas as pl

_D = 256
_H = 8
_HD = 32
_N = 1024
_NE = 1024
_AR = 8
_NT = 32
_ET = 16


def _gelu(x):
  return 0.5 * x * (1.0 + jax.lax.erf(x * (1.0 / math.sqrt(2.0))))


def _ln(x, g, b):
  m = jnp.mean(x, axis=-1, keepdims=True)
  v = jnp.mean((x - m) ** 2, axis=-1, keepdims=True)
  return (x - m) * jax.lax.rsqrt(v + 1e-5) * g + b


def _mm(a, b):
  return jnp.dot(a, b, preferred_element_type=jnp.float32)


def _rel(x, w):
  h = _mm(x, w['node_proj_w']) + w['node_proj_b']
  q = _mm(h, w['query_w']) + w['query_b']
  k = _mm(h, w['key_w']) + w['key_b']
  v = _mm(h, w['value_w']) + w['value_b']
  outs = []
  for hh in range(_H):
    sl = slice(hh * _HD, (hh + 1) * _HD)
    s = jax.lax.dot_general(q[:, sl], k[:, sl], (((1,), (1,)), ((), ())),
                            preferred_element_type=jnp.float32)
    s = s * (1.0 / math.sqrt(_HD))
    m = jnp.max(s, axis=-1, keepdims=True)
    e = jnp.exp(s - m)
    p = e / jnp.sum(e, axis=-1, keepdims=True)
    outs.append(_mm(p, v[:, sl]))
  attn_out = jnp.concatenate(outs, axis=-1)
  o = _mm(attn_out, w['out_proj_w']) + w['out_proj_b'] + x
  o = _ln(o, w['norm_g'], w['norm_b'])
  ff = _mm(_gelu(_mm(o, w['ff1_w']) + w['ff1_b']), w['ff2_w']) + w['ff2_b']
  return _ln(ff + o, w['ffn_g'], w['ffn_b'])


def _hyp(x, w_inc, cnt, counts, edge_emb, hw):
  pooled = jax.lax.dot_general(w_inc, x, (((1,), (0,)), ((), ())),
                               preferred_element_type=jnp.float32) / cnt
  ef = _mm(pooled, hw['enc_w1']) + _mm(edge_emb, hw['enc_w2']) + hw['enc_b']
  ef = _ln(_gelu(ef), hw['enc_g'], hw['enc_bb'])
  nup = jax.lax.dot_general(w_inc, ef, (((0,), (0,)), ((), ())),
                            preferred_element_type=jnp.float32) / counts
  u = _mm(x, hw['upd_w1']) + _mm(nup, hw['upd_w2']) + hw['upd_b']
  return _ln(_gelu(u), hw['upd_g'], hw['upd_bb'])


_REL_KEYS = ['node_proj_w', 'node_proj_b', 'query_w', 'query_b',
             'key_w', 'key_b', 'value_w', 'value_b',
             'out_proj_w', 'out_proj_b', 'norm_g', 'norm_b',
             'ff1_w', 'ff1_b', 'ff2_w', 'ff2_b', 'ffn_g', 'ffn_b']
_HYP_KEYS = ['he_etype', 'enc_w1', 'enc_w2', 'enc_b', 'enc_g', 'enc_bb',
             'upd_w1', 'upd_w2', 'upd_b', 'upd_g', 'upd_bb']
_L = 2
_NW = len(_REL_KEYS) + len(_HYP_KEYS)


def _read_named(refs, keys):
  return {k: r[...] for k, r in zip(keys, refs)}


def _mega_body(ids_ref, emb_ref, maskf_ref, mem_ref, types_ref, *rest):
  lw = []
  for li in range(_L):
    base = li * _NW
    rel_refs = rest[base:base + len(_REL_KEYS)]
    hyp_refs = rest[base + len(_REL_KEYS):base + _NW]
    lw.append((_read_named(rel_refs, _REL_KEYS),
               _read_named(hyp_refs, _HYP_KEYS)))
  pw_ref, pb_ref, pg_ref, pbb_ref, ox_ref, og_ref = rest[_L * _NW:]

  ids = ids_ref[0]
  tid = jax.lax.broadcasted_iota(jnp.int32, (_NT, _N), 0)
  oh = jnp.where(ids == tid, 1.0, 0.0)
  x = jax.lax.dot_general(oh, emb_ref[...], (((0,), (0,)), ((), ())),
                          preferred_element_type=jnp.float32)

  maskf = maskf_ref[0]
  mem = mem_ref[0]
  nid = jax.lax.broadcasted_iota(jnp.int32, (_NE, _N), 1)
  w_inc = jnp.zeros((_NE, _N), jnp.float32)
  for a in range(_AR):
    hit = mem[:, a:a + 1] == nid
    w_inc = w_inc + jnp.where(hit, maskf[:, a:a + 1], 0.0)
  cnt = jnp.clip(jnp.sum(maskf, axis=-1, keepdims=True), 1.0)
  counts = jax.lax.dot_general(w_inc, jnp.ones((_NE, 1), jnp.float32),
                               (((0,), (0,)), ((), ())),
                               preferred_element_type=jnp.float32)
  counts = jnp.clip(counts, 1.0)

  types = types_ref[0]
  eid = jax.lax.broadcasted_iota(jnp.int32, (_ET, _NE), 0)
  eoh = jnp.where(types == eid, 1.0, 0.0)

  for rw, hw in lw:
    x = _rel(x, rw)
    edge_emb = jax.lax.dot_general(eoh, hw['he_etype'],
                                   (((0,), (0,)), ((), ())),
                                   preferred_element_type=jnp.float32)
    x = _hyp(x, w_inc, cnt, counts, edge_emb, hw)

  ox_ref[0] = x
  gm = jnp.mean(x, axis=0, keepdims=True)
  gm = _mm(gm, pw_ref[...]) + pb_ref[...]
  og_ref[0] = _ln(_gelu(gm), pg_ref[...], pbb_ref[...])


def _layer_weight_args(p):
  out = []
  for k in _REL_KEYS:
    a = p[k]
    out.append(a.reshape(1, -1) if a.ndim == 1 else a)
  out += [p['he_etype'], p['enc_w'][:_D], p['enc_w'][_D:],
          p['enc_b'].reshape(1, _D), p['enc_g'].reshape(1, _D),
          p['enc_bb'].reshape(1, _D),
          p['upd_w'][:_D], p['upd_w'][_D:], p['upd_b'].reshape(1, _D),
          p['upd_g'].reshape(1, _D), p['upd_bb'].reshape(1, _D)]
  return out


def _full_spec(arr):
  nd = arr.ndim
  return pl.BlockSpec(arr.shape, lambda b, _n=nd: (0,) * _n)


def kernel(node_type_ids, edge_index, edge_types, hyperedge_members,
           hyperedge_types, hyperedge_mask, params):
  del edge_index, edge_types
  B = node_type_ids.shape[0]

  ids = node_type_ids.astype(jnp.int32)
  members = hyperedge_members.astype(jnp.int32)
  types = hyperedge_types.astype(jnp.int32)
  maskf = hyperedge_mask.astype(jnp.float32)

  args = [ids.reshape(B, 1, _N), params['node_type_embed'], maskf, members,
          types.reshape(B, 1, _NE)]
  for p in params['layers']:
    args += _layer_weight_args(p)
  args += [params['pool_w'], params['pool_b'].reshape(1, _D),
           params['pool_g'].reshape(1, _D), params['pool_bb'].reshape(1, _D)]

  in_specs = [
      pl.BlockSpec((1, 1, _N), lambda b: (b, 0, 0)),
      _full_spec(params['node_type_embed']),
      pl.BlockSpec((1, _NE, _AR), lambda b: (b, 0, 0)),
      pl.BlockSpec((1, _NE, _AR), lambda b: (b, 0, 0)),
      pl.BlockSpec((1, 1, _NE), lambda b: (b, 0, 0)),
  ] + [_full_spec(a) for a in args[5:]]

  x_out, graph_emb = pl.pallas_call(
      _mega_body,
      grid=(B,),
      in_specs=in_specs,
      out_specs=[
          pl.BlockSpec((1, _N, _D), lambda b: (b, 0, 0)),
          pl.BlockSpec((1, 1, _D), lambda b: (b, 0, 0)),
      ],
      out_shape=[
          jax.ShapeDtypeStruct((B, _N, _D), jnp.float32),
          jax.ShapeDtypeStruct((B, 1, _D), jnp.float32),
      ],
  )(*args)
  return x_out, graph_emb.reshape(B, _D)


# speedup vs baseline: 3.2591x; 1.0727x over previous
import math

import jax
import jax.numpy as jnp
from jax.experimental import pallas as pl

_D = 256
_H = 8
_HD = 32
_N = 1024
_NE = 1024
_AR = 8
_NT = 32
_ET = 16


def _gelu(x):
  return 0.5 * x * (1.0 + jax.lax.erf(x * (1.0 / math.sqrt(2.0))))


def _ln(x, g, b):
  m = jnp.mean(x, axis=-1, keepdims=True)
  v = jnp.mean((x - m) ** 2, axis=-1, keepdims=True)
  return (x - m) * jax.lax.rsqrt(v + 1e-5) * g + b


def _mm(a, b):
  return jnp.dot(a, b, preferred_element_type=jnp.float32)


def _rel(x, w):
  h = _mm(x, w['node_proj_w']) + w['node_proj_b']
  q = _mm(h, w['query_w']) + w['query_b']
  k = _mm(h, w['key_w']) + w['key_b']
  v = _mm(h, w['value_w']) + w['value_b']
  outs = []
  for hh in range(_H):
    sl = slice(hh * _HD, (hh + 1) * _HD)
    s = jax.lax.dot_general(q[:, sl].astype(jnp.bfloat16),
                            k[:, sl].astype(jnp.bfloat16),
                            (((1,), (1,)), ((), ())),
                            preferred_element_type=jnp.float32)
    s = s * (1.0 / math.sqrt(_HD))
    m = jnp.max(s, axis=-1, keepdims=True)
    e = jnp.exp(s - m)
    p = e / jnp.sum(e, axis=-1, keepdims=True)
    outs.append(jax.lax.dot_general(p.astype(jnp.bfloat16),
                                    v[:, sl].astype(jnp.bfloat16),
                                    (((1,), (0,)), ((), ())),
                                    preferred_element_type=jnp.float32))
  attn_out = jnp.concatenate(outs, axis=-1)
  o = _mm(attn_out, w['out_proj_w']) + w['out_proj_b'] + x
  o = _ln(o, w['norm_g'], w['norm_b'])
  ff = _mm(_gelu(_mm(o, w['ff1_w']) + w['ff1_b']), w['ff2_w']) + w['ff2_b']
  return _ln(ff + o, w['ffn_g'], w['ffn_b'])


def _hyp(x, w_inc, cnt, counts, edge_emb, hw):
  pooled = jax.lax.dot_general(w_inc, x, (((1,), (0,)), ((), ())),
                               preferred_element_type=jnp.float32) / cnt
  ef = _mm(pooled, hw['enc_w1']) + _mm(edge_emb, hw['enc_w2']) + hw['enc_b']
  ef = _ln(_gelu(ef), hw['enc_g'], hw['enc_bb'])
  nup = jax.lax.dot_general(w_inc, ef, (((0,), (0,)), ((), ())),
                            preferred_element_type=jnp.float32) / counts
  u = _mm(x, hw['upd_w1']) + _mm(nup, hw['upd_w2']) + hw['upd_b']
  return _ln(_gelu(u), hw['upd_g'], hw['upd_bb'])


_REL_KEYS = ['node_proj_w', 'node_proj_b', 'query_w', 'query_b',
             'key_w', 'key_b', 'value_w', 'value_b',
             'out_proj_w', 'out_proj_b', 'norm_g', 'norm_b',
             'ff1_w', 'ff1_b', 'ff2_w', 'ff2_b', 'ffn_g', 'ffn_b']
_HYP_KEYS = ['he_etype', 'enc_w1', 'enc_w2', 'enc_b', 'enc_g', 'enc_bb',
             'upd_w1', 'upd_w2', 'upd_b', 'upd_g', 'upd_bb']
_L = 2
_NW = len(_REL_KEYS) + len(_HYP_KEYS)


def _read_named(refs, keys):
  return {k: r[...] for k, r in zip(keys, refs)}


def _mega_body(ids_ref, emb_ref, maskf_ref, mem_ref, types_ref, *rest):
  lw = []
  for li in range(_L):
    base = li * _NW
    rel_refs = rest[base:base + len(_REL_KEYS)]
    hyp_refs = rest[base + len(_REL_KEYS):base + _NW]
    lw.append((_read_named(rel_refs, _REL_KEYS),
               _read_named(hyp_refs, _HYP_KEYS)))
  pw_ref, pb_ref, pg_ref, pbb_ref, ox_ref, og_ref = rest[_L * _NW:]

  ids = ids_ref[0]
  tid = jax.lax.broadcasted_iota(jnp.int32, (_NT, _N), 0)
  oh = jnp.where(ids == tid, 1.0, 0.0)
  x = jax.lax.dot_general(oh, emb_ref[...], (((0,), (0,)), ((), ())),
                          preferred_element_type=jnp.float32)

  maskf = maskf_ref[0]
  mem = mem_ref[0]
  nid = jax.lax.broadcasted_iota(jnp.int32, (_NE, _N), 1)
  w_inc = jnp.zeros((_NE, _N), jnp.float32)
  for a in range(_AR):
    hit = mem[:, a:a + 1] == nid
    w_inc = w_inc + jnp.where(hit, maskf[:, a:a + 1], 0.0)
  cnt = jnp.clip(jnp.sum(maskf, axis=-1, keepdims=True), 1.0)
  counts = jax.lax.dot_general(w_inc, jnp.ones((_NE, 1), jnp.float32),
                               (((0,), (0,)), ((), ())),
                               preferred_element_type=jnp.float32)
  counts = jnp.clip(counts, 1.0)

  types = types_ref[0]
  eid = jax.lax.broadcasted_iota(jnp.int32, (_ET, _NE), 0)
  eoh = jnp.where(types == eid, 1.0, 0.0)

  for rw, hw in lw:
    x = _rel(x, rw)
    edge_emb = jax.lax.dot_general(eoh, hw['he_etype'],
                                   (((0,), (0,)), ((), ())),
                                   preferred_element_type=jnp.float32)
    x = _hyp(x, w_inc, cnt, counts, edge_emb, hw)

  ox_ref[0] = x
  gm = jnp.mean(x, axis=0, keepdims=True)
  gm = _mm(gm, pw_ref[...]) + pb_ref[...]
  og_ref[0] = _ln(_gelu(gm), pg_ref[...], pbb_ref[...])


def _layer_weight_args(p):
  out = []
  for k in _REL_KEYS:
    a = p[k]
    out.append(a.reshape(1, -1) if a.ndim == 1 else a)
  out += [p['he_etype'], p['enc_w'][:_D], p['enc_w'][_D:],
          p['enc_b'].reshape(1, _D), p['enc_g'].reshape(1, _D),
          p['enc_bb'].reshape(1, _D),
          p['upd_w'][:_D], p['upd_w'][_D:], p['upd_b'].reshape(1, _D),
          p['upd_g'].reshape(1, _D), p['upd_bb'].reshape(1, _D)]
  return out


def _full_spec(arr):
  nd = arr.ndim
  return pl.BlockSpec(arr.shape, lambda b, _n=nd: (0,) * _n)


def kernel(node_type_ids, edge_index, edge_types, hyperedge_members,
           hyperedge_types, hyperedge_mask, params):
  del edge_index, edge_types
  B = node_type_ids.shape[0]

  ids = node_type_ids.astype(jnp.int32)
  members = hyperedge_members.astype(jnp.int32)
  types = hyperedge_types.astype(jnp.int32)
  maskf = hyperedge_mask.astype(jnp.float32)

  args = [ids.reshape(B, 1, _N), params['node_type_embed'], maskf, members,
          types.reshape(B, 1, _NE)]
  for p in params['layers']:
    args += _layer_weight_args(p)
  args += [params['pool_w'], params['pool_b'].reshape(1, _D),
           params['pool_g'].reshape(1, _D), params['pool_bb'].reshape(1, _D)]

  in_specs = [
      pl.BlockSpec((1, 1, _N), lambda b: (b, 0, 0)),
      _full_spec(params['node_type_embed']),
      pl.BlockSpec((1, _NE, _AR), lambda b: (b, 0, 0)),
      pl.BlockSpec((1, _NE, _AR), lambda b: (b, 0, 0)),
      pl.BlockSpec((1, 1, _NE), lambda b: (b, 0, 0)),
  ] + [_full_spec(a) for a in args[5:]]

  x_out, graph_emb = pl.pallas_call(
      _mega_body,
      grid=(B,),
      in_specs=in_specs,
      out_specs=[
          pl.BlockSpec((1, _N, _D), lambda b: (b, 0, 0)),
          pl.BlockSpec((1, 1, _D), lambda b: (b, 0, 0)),
      ],
      out_shape=[
          jax.ShapeDtypeStruct((B, _N, _D), jnp.float32),
          jax.ShapeDtypeStruct((B, 1, _D), jnp.float32),
      ],
  )(*args)
  return x_out, graph_emb.reshape(B, _D)
